# baseline (speedup 1.0000x reference)
import functools
import jax
import jax.numpy as jnp
from jax import lax
from jax.experimental import pallas as pl
from jax.experimental.pallas import tpu as pltpu, tpu_sc as plsc

SIGMA = 1.0
EPS = 1e-5

_CHUNK = 128


def _sc_gather(t1, t2, idx2d):
    n_rows = idx2d.shape[0] * idx2d.shape[1]
    D1, D2 = t1.shape[1], t2.shape[1]
    info = plsc.get_sparse_core_info()
    nw = info.num_cores * info.num_subcores
    cpw = idx2d.shape[0] // nw

    @functools.partial(
        pl.kernel,
        mesh=plsc.VectorSubcoreMesh(core_axis_name="c", subcore_axis_name="s"),
        out_type=[
            jax.ShapeDtypeStruct((n_rows, D1), jnp.float32),
            jax.ShapeDtypeStruct((n_rows, D2), jnp.float32),
        ],
        scratch_types=[
            pltpu.VMEM((cpw, _CHUNK), jnp.int32),
            pltpu.VMEM((_CHUNK, D1), jnp.float32),
            pltpu.VMEM((_CHUNK, D1), jnp.float32),
            pltpu.VMEM((_CHUNK, D2), jnp.float32),
            pltpu.VMEM((_CHUNK, D2), jnp.float32),
            pltpu.SemaphoreType.DMA,
            pltpu.SemaphoreType.DMA,
        ],
    )
    def gather(t1_h, t2_h, idx_h, o1_h, o2_h,
               idx_v, b1a, b1b, b2a, b2b, sema, semb):
        wid = lax.axis_index("s") * info.num_cores + lax.axis_index("c")
        pltpu.sync_copy(idx_h.at[pl.ds(wid * cpw, cpw)], idx_v)

        def fire(j, u1, u2, sem):
            pltpu.make_async_copy(t1_h.at[idx_v.at[j]], u1, sem).start()
            pltpu.make_async_copy(t2_h.at[idx_v.at[j]], u2, sem).start()

        def drain(j, u1, u2, sem):
            pltpu.make_async_copy(t1_h.at[idx_v.at[j]], u1, sem).wait()
            pltpu.make_async_copy(t2_h.at[idx_v.at[j]], u2, sem).wait()
            row0 = (wid * cpw + j) * _CHUNK
            pltpu.sync_copy(u1, o1_h.at[pl.ds(row0, _CHUNK)])
            pltpu.sync_copy(u2, o2_h.at[pl.ds(row0, _CHUNK)])

        fire(0, b1a, b2a, sema)

        def pair(i, carry):
            fire(2 * i + 1, b1b, b2b, semb)
            drain(2 * i, b1a, b2a, sema)

            @pl.when(i < cpw // 2 - 1)
            def _():
                fire(2 * i + 2, b1a, b2a, sema)

            drain(2 * i + 1, b1b, b2b, semb)
            return carry

        lax.fori_loop(0, cpw // 2, pair, 0)

    return gather(t1, t2, idx2d)


def _k1_body(x_ref, pt_ref, w0_ref, b0_ref, y0t_ref, xt_ref, s1_ref, s2_ref):
    b = pl.program_id(0)
    j = pl.program_id(1)
    xb = x_ref[0]
    w0 = w0_ref[...]
    y = lax.dot_general(xb, w0, (((0,), (1,)), ((), ())),
                        preferred_element_type=jnp.float32)
    y = y + b0_ref[...]
    nblk, cmid = y.shape
    pad = jnp.zeros((nblk, 128 - cmid - 3), jnp.float32)
    y0t_ref[0] = jnp.concatenate([y, pt_ref[0], pad], axis=1)
    xt_ref[0] = xb.T

    @pl.when((b == 0) & (j == 0))
    def _():
        s1_ref[...] = jnp.zeros_like(s1_ref)
        s2_ref[...] = jnp.zeros_like(s2_ref)

    s1_ref[...] += jnp.sum(y, axis=0, keepdims=True)
    s2_ref[...] += jnp.sum(y * y, axis=0, keepdims=True)


def _conv0_stats(x, post, W0, b0, nblk=512):
    B, CIN, N = x.shape
    CMID = W0.shape[0]
    grid = (B, N // nblk)
    return pl.pallas_call(
        _k1_body,
        grid=grid,
        in_specs=[
            pl.BlockSpec((1, CIN, nblk), lambda b, j: (b, 0, j)),
            pl.BlockSpec((1, nblk, 3), lambda b, j: (b, j, 0)),
            pl.BlockSpec((CMID, CIN), lambda b, j: (0, 0)),
            pl.BlockSpec((1, CMID), lambda b, j: (0, 0)),
        ],
        out_specs=[
            pl.BlockSpec((1, nblk, 128), lambda b, j: (b, j, 0)),
            pl.BlockSpec((1, nblk, CIN), lambda b, j: (b, j, 0)),
            pl.BlockSpec((1, CMID), lambda b, j: (0, 0)),
            pl.BlockSpec((1, CMID), lambda b, j: (0, 0)),
        ],
        out_shape=[
            jax.ShapeDtypeStruct((B, N, 128), jnp.float32),
            jax.ShapeDtypeStruct((B, N, CIN), jnp.float32),
            jax.ShapeDtypeStruct((1, CMID), jnp.float32),
            jax.ShapeDtypeStruct((1, CMID), jnp.float32),
        ],
    )(x, post, W0, b0.reshape(1, -1))


def _k2_body(f_ref, sp_ref, m_ref, kpt_ref, wk_ref,
             a0_ref, c0_ref, bk_ref, y1t_ref, s1_ref, s2_ref):
    b = pl.program_id(0)
    j = pl.program_id(1)
    KP_N = wk_ref.shape[0]
    CMID = wk_ref.shape[1]
    g = f_ref[0]
    sblk, K = g.shape[0], g.shape[1]
    f_raw = lax.slice(g, (0, 0, 0), (sblk, K, CMID))
    a0 = a0_ref[...][:, None, :]
    c0 = c0_ref[...][:, None, :]
    f = jnp.maximum(f_raw * a0 + c0, 0.0)

    kx = kpt_ref[0][None, None, :]
    ky = kpt_ref[1][None, None, :]
    kz = kpt_ref[2][None, None, :]
    sp = sp_ref[0]
    rx = (lax.slice(g, (0, 0, CMID), (sblk, K, CMID + 1))
          - lax.slice(sp, (0, 0), (sblk, 1))[:, :, None])
    ry = (lax.slice(g, (0, 0, CMID + 1), (sblk, K, CMID + 2))
          - lax.slice(sp, (0, 1), (sblk, 2))[:, :, None])
    rz = (lax.slice(g, (0, 0, CMID + 2), (sblk, K, CMID + 3))
          - lax.slice(sp, (0, 2), (sblk, 3))[:, :, None])
    d2 = (rx - kx) ** 2 + (ry - ky) ** 2 + (rz - kz) ** 2
    d = jnp.sqrt(d2 + 1e-12)
    w = jnp.maximum(1.0 - d / SIGMA, 0.0) * m_ref[0][:, :, None]

    gpc = lax.dot_general(w, f, (((1,), (1,)), ((0,), (0,))),
                          preferred_element_type=jnp.float32)
    acc = jnp.zeros((sblk, CMID), jnp.float32)
    for p in range(KP_N):
        gp = lax.slice(gpc, (0, p, 0), (sblk, p + 1, CMID)).reshape(sblk, CMID)
        acc = acc + jnp.dot(gp, wk_ref[p],
                            preferred_element_type=jnp.float32)
    y1 = acc + bk_ref[...]
    y1t_ref[0] = y1

    @pl.when((b == 0) & (j == 0))
    def _():
        s1_ref[...] = jnp.zeros_like(s1_ref)
        s2_ref[...] = jnp.zeros_like(s2_ref)

    s1_ref[...] += jnp.sum(y1, axis=0, keepdims=True)
    s2_ref[...] += jnp.sum(y1 * y1, axis=0, keepdims=True)


def _point_conv(g1, spt, maskf, KPt, Wk, a0, c0, bk, sblk=256):
    B, S, K, _ = g1.shape
    CMID = Wk.shape[1]
    KP_N = Wk.shape[0]
    grid = (B, S // sblk)
    return pl.pallas_call(
        _k2_body,
        grid=grid,
        in_specs=[
            pl.BlockSpec((1, sblk, K, 128), lambda b, j: (b, j, 0, 0)),
            pl.BlockSpec((1, sblk, 3), lambda b, j: (b, j, 0)),
            pl.BlockSpec((1, sblk, K), lambda b, j: (b, j, 0)),
            pl.BlockSpec((3, KP_N), lambda b, j: (0, 0)),
            pl.BlockSpec((KP_N, CMID, CMID), lambda b, j: (0, 0, 0)),
            pl.BlockSpec((1, CMID), lambda b, j: (0, 0)),
            pl.BlockSpec((1, CMID), lambda b, j: (0, 0)),
            pl.BlockSpec((1, CMID), lambda b, j: (0, 0)),
        ],
        out_specs=[
            pl.BlockSpec((1, sblk, CMID), lambda b, j: (b, j, 0)),
            pl.BlockSpec((1, CMID), lambda b, j: (0, 0)),
            pl.BlockSpec((1, CMID), lambda b, j: (0, 0)),
        ],
        out_shape=[
            jax.ShapeDtypeStruct((B, S, CMID), jnp.float32),
            jax.ShapeDtypeStruct((1, CMID), jnp.float32),
            jax.ShapeDtypeStruct((1, CMID), jnp.float32),
        ],
    )(g1, spt, maskf, KPt, Wk,
      a0.reshape(1, -1), c0.reshape(1, -1), bk.reshape(1, -1))


def _k3_body(y1_ref, xg_ref, m_ref, w2_ref, a1_ref, c1_ref, b2_ref,
             y2t_ref, xs_ref, s1_ref, s2_ref):
    b = pl.program_id(0)
    j = pl.program_id(1)
    h1 = jnp.maximum(y1_ref[0] * a1_ref[...] + c1_ref[...], 0.0)
    y2 = lax.dot_general(h1, w2_ref[...], (((1,), (1,)), ((), ())),
                         preferred_element_type=jnp.float32)
    y2 = y2 + b2_ref[...]
    y2t_ref[0] = y2

    xg = xg_ref[0]
    m = m_ref[0][:, :, None]
    xm = jnp.where(m > 0.0, xg, -1e9)
    xs = jnp.max(xm, axis=1)
    xs = jnp.where(xs <= -1e9 + 1.0, 0.0, xs)
    xs_ref[0] = xs

    @pl.when((b == 0) & (j == 0))
    def _():
        s1_ref[...] = jnp.zeros_like(s1_ref)
        s2_ref[...] = jnp.zeros_like(s2_ref)

    s1_ref[...] += jnp.sum(y2, axis=0, keepdims=True)
    s2_ref[...] += jnp.sum(y2 * y2, axis=0, keepdims=True)


def _conv2_pool(y1t, xg, maskf, W2, a1, c1, b2, sblk=256):
    B, S, CMID = y1t.shape
    COUT = W2.shape[0]
    K = xg.shape[2]
    grid = (B, S // sblk)
    return pl.pallas_call(
        _k3_body,
        grid=grid,
        in_specs=[
            pl.BlockSpec((1, sblk, CMID), lambda b, j: (b, j, 0)),
            pl.BlockSpec((1, sblk, K, COUT), lambda b, j: (b, j, 0, 0)),
            pl.BlockSpec((1, sblk, K), lambda b, j: (b, j, 0)),
            pl.BlockSpec((COUT, CMID), lambda b, j: (0, 0)),
            pl.BlockSpec((1, CMID), lambda b, j: (0, 0)),
            pl.BlockSpec((1, CMID), lambda b, j: (0, 0)),
            pl.BlockSpec((1, COUT), lambda b, j: (0, 0)),
        ],
        out_specs=[
            pl.BlockSpec((1, sblk, COUT), lambda b, j: (b, j, 0)),
            pl.BlockSpec((1, sblk, COUT), lambda b, j: (b, j, 0)),
            pl.BlockSpec((1, COUT), lambda b, j: (0, 0)),
            pl.BlockSpec((1, COUT), lambda b, j: (0, 0)),
        ],
        out_shape=[
            jax.ShapeDtypeStruct((B, S, COUT), jnp.float32),
            jax.ShapeDtypeStruct((B, S, COUT), jnp.float32),
            jax.ShapeDtypeStruct((1, COUT), jnp.float32),
            jax.ShapeDtypeStruct((1, COUT), jnp.float32),
        ],
    )(y1t, xg, maskf, W2, a1.reshape(1, -1), c1.reshape(1, -1),
      b2.reshape(1, -1))


def _k4_body(y2_ref, xs_ref, a2_ref, c2_ref, out_ref):
    y = y2_ref[0] * a2_ref[...] + c2_ref[...] + xs_ref[0]
    out_ref[0] = jnp.maximum(y, 0.0).T


def _finalize(y2t, xs, a2, c2, sblk=512):
    B, S, COUT = y2t.shape
    grid = (B, S // sblk)
    return pl.pallas_call(
        _k4_body,
        grid=grid,
        in_specs=[
            pl.BlockSpec((1, sblk, COUT), lambda b, j: (b, j, 0)),
            pl.BlockSpec((1, sblk, COUT), lambda b, j: (b, j, 0)),
            pl.BlockSpec((1, COUT), lambda b, j: (0, 0)),
            pl.BlockSpec((1, COUT), lambda b, j: (0, 0)),
        ],
        out_specs=pl.BlockSpec((1, COUT, sblk), lambda b, j: (b, 0, j)),
        out_shape=jax.ShapeDtypeStruct((B, COUT, S), jnp.float32),
    )(y2t, xs, a2.reshape(1, -1), c2.reshape(1, -1))


def _bn_affine(s1, s2, cnt, g, be):
    m = s1.reshape(-1) / cnt
    v = s2.reshape(-1) / cnt - m * m
    a = g * lax.rsqrt(v + EPS)
    c = be - m * a
    return a, c


@jax.jit
def kernel(x, pos, support_points, neighbors_indices, mask_indices,
           W0, b0, g0, be0, KP, Wk, bk, g1, be1, W2, b2, g2, be2):
    B, CIN, N = x.shape
    S, K = neighbors_indices.shape[1:]
    maskf = mask_indices.astype(jnp.float32)

    post = jnp.transpose(pos, (0, 2, 1))
    y0t, xt, s1_0, s2_0 = _conv0_stats(x, post, W0, b0)
    a0, c0 = _bn_affine(s1_0, s2_0, B * N, g0, be0)

    nbr = neighbors_indices
    idx_flat = (nbr + (jnp.arange(B, dtype=nbr.dtype) * N)[:, None, None])
    idx2d = idx_flat.reshape(-1, _CHUNK)
    gf, gx = _sc_gather(y0t.reshape(B * N, 128),
                        xt.reshape(B * N, CIN), idx2d)
    gf = gf.reshape(B, S, K, 128)
    xg = gx.reshape(B, S, K, CIN)
    spt = jnp.transpose(support_points, (0, 2, 1))

    y1t, s1_1, s2_1 = _point_conv(gf, spt, maskf, KP.T, Wk, a0, c0, bk)
    a1, c1 = _bn_affine(s1_1, s2_1, B * S, g1, be1)

    y2t, xs, s1_2, s2_2 = _conv2_pool(y1t, xg, maskf, W2, a1, c1, b2)
    a2, c2 = _bn_affine(s1_2, s2_2, B * S, g2, be2)

    out = _finalize(y2t, xs, a2, c2)
    return (out, support_points, neighbors_indices, mask_indices)


# speedup vs baseline: 1.1286x; 1.1286x over previous
import functools
import jax
import jax.numpy as jnp
from jax import lax
from jax.experimental import pallas as pl
from jax.experimental.pallas import tpu as pltpu, tpu_sc as plsc

SIGMA = 1.0
EPS = 1e-5

_CHUNK = 128


def _sc_gather(t, idx2d):
    n_rows = idx2d.shape[0] * idx2d.shape[1]
    D = t.shape[1]
    info = plsc.get_sparse_core_info()
    nw = info.num_cores * info.num_subcores
    cpw = idx2d.shape[0] // nw

    @functools.partial(
        pl.kernel,
        mesh=plsc.VectorSubcoreMesh(core_axis_name="c", subcore_axis_name="s"),
        out_type=jax.ShapeDtypeStruct((n_rows, D), jnp.float32),
        scratch_types=[
            pltpu.VMEM((cpw, _CHUNK), jnp.int32),
            pltpu.VMEM((_CHUNK, D), jnp.float32),
            pltpu.VMEM((_CHUNK, D), jnp.float32),
            pltpu.SemaphoreType.DMA,
            pltpu.SemaphoreType.DMA,
        ],
    )
    def gather(t_h, idx_h, o_h, idx_v, ba, bb, sema, semb):
        wid = lax.axis_index("s") * info.num_cores + lax.axis_index("c")
        pltpu.sync_copy(idx_h.at[pl.ds(wid * cpw, cpw)], idx_v)

        def fire(j, buf, sem):
            pltpu.make_async_copy(t_h.at[idx_v.at[j]], buf, sem).start()

        def drain(j, buf, sem):
            pltpu.make_async_copy(t_h.at[idx_v.at[j]], buf, sem).wait()
            row0 = (wid * cpw + j) * _CHUNK
            pltpu.sync_copy(buf, o_h.at[pl.ds(row0, _CHUNK)])

        fire(0, ba, sema)

        def pair(i, carry):
            fire(2 * i + 1, bb, semb)
            drain(2 * i, ba, sema)

            @pl.when(i < cpw // 2 - 1)
            def _():
                fire(2 * i + 2, ba, sema)

            drain(2 * i + 1, bb, semb)
            return carry

        lax.fori_loop(0, cpw // 2, pair, 0)

    return gather(t, idx2d)


def _k1_body(x_ref, pt_ref, w0_ref, b0_ref, y0t_ref, xt_ref, s1_ref, s2_ref):
    b = pl.program_id(0)
    j = pl.program_id(1)
    xb = x_ref[0]
    w0 = w0_ref[...]
    y = lax.dot_general(xb, w0, (((0,), (1,)), ((), ())),
                        preferred_element_type=jnp.float32)
    y = y + b0_ref[...]
    nblk, cmid = y.shape
    pad = jnp.zeros((nblk, 128 - cmid - 3), jnp.float32)
    y0t_ref[0] = jnp.concatenate([y, pt_ref[0], pad], axis=1)
    xt_ref[0] = xb.T

    @pl.when((b == 0) & (j == 0))
    def _():
        s1_ref[...] = jnp.zeros_like(s1_ref)
        s2_ref[...] = jnp.zeros_like(s2_ref)

    s1_ref[...] += jnp.sum(y, axis=0, keepdims=True)
    s2_ref[...] += jnp.sum(y * y, axis=0, keepdims=True)


def _conv0_stats(x, post, W0, b0, nblk=512):
    B, CIN, N = x.shape
    CMID = W0.shape[0]
    grid = (B, N // nblk)
    return pl.pallas_call(
        _k1_body,
        grid=grid,
        in_specs=[
            pl.BlockSpec((1, CIN, nblk), lambda b, j: (b, 0, j)),
            pl.BlockSpec((1, nblk, 3), lambda b, j: (b, j, 0)),
            pl.BlockSpec((CMID, CIN), lambda b, j: (0, 0)),
            pl.BlockSpec((1, CMID), lambda b, j: (0, 0)),
        ],
        out_specs=[
            pl.BlockSpec((1, nblk, 128), lambda b, j: (b, j, 0)),
            pl.BlockSpec((1, nblk, CIN), lambda b, j: (b, j, 0)),
            pl.BlockSpec((1, CMID), lambda b, j: (0, 0)),
            pl.BlockSpec((1, CMID), lambda b, j: (0, 0)),
        ],
        out_shape=[
            jax.ShapeDtypeStruct((B, N, 128), jnp.float32),
            jax.ShapeDtypeStruct((B, N, CIN), jnp.float32),
            jax.ShapeDtypeStruct((1, CMID), jnp.float32),
            jax.ShapeDtypeStruct((1, CMID), jnp.float32),
        ],
    )(x, post, W0, b0.reshape(1, -1))


def _k2_body(f_ref, sp_ref, m_ref, kpt_ref, wk_ref,
             a0_ref, c0_ref, bk_ref, y1t_ref, s1_ref, s2_ref):
    b = pl.program_id(0)
    j = pl.program_id(1)
    KP_N = wk_ref.shape[0]
    CMID = wk_ref.shape[1]
    g = f_ref[0]
    sblk, K = g.shape[0], g.shape[1]
    f_raw = lax.slice(g, (0, 0, 0), (sblk, K, CMID))
    a0 = a0_ref[...][:, None, :]
    c0 = c0_ref[...][:, None, :]
    f = jnp.maximum(f_raw * a0 + c0, 0.0)

    kx = kpt_ref[0][None, None, :]
    ky = kpt_ref[1][None, None, :]
    kz = kpt_ref[2][None, None, :]
    sp = sp_ref[0]
    rx = (lax.slice(g, (0, 0, CMID), (sblk, K, CMID + 1))
          - lax.slice(sp, (0, 0), (sblk, 1))[:, :, None])
    ry = (lax.slice(g, (0, 0, CMID + 1), (sblk, K, CMID + 2))
          - lax.slice(sp, (0, 1), (sblk, 2))[:, :, None])
    rz = (lax.slice(g, (0, 0, CMID + 2), (sblk, K, CMID + 3))
          - lax.slice(sp, (0, 2), (sblk, 3))[:, :, None])
    d2 = (rx - kx) ** 2 + (ry - ky) ** 2 + (rz - kz) ** 2
    d = jnp.sqrt(d2 + 1e-12)
    w = jnp.maximum(1.0 - d / SIGMA, 0.0) * m_ref[0][:, :, None]

    gpc = lax.dot_general(w, f, (((1,), (1,)), ((0,), (0,))),
                          preferred_element_type=jnp.float32)
    acc = jnp.zeros((sblk, CMID), jnp.float32)
    for p in range(KP_N):
        gp = lax.slice(gpc, (0, p, 0), (sblk, p + 1, CMID)).reshape(sblk, CMID)
        acc = acc + jnp.dot(gp, wk_ref[p],
                            preferred_element_type=jnp.float32)
    y1 = acc + bk_ref[...]
    y1t_ref[0] = y1

    @pl.when((b == 0) & (j == 0))
    def _():
        s1_ref[...] = jnp.zeros_like(s1_ref)
        s2_ref[...] = jnp.zeros_like(s2_ref)

    s1_ref[...] += jnp.sum(y1, axis=0, keepdims=True)
    s2_ref[...] += jnp.sum(y1 * y1, axis=0, keepdims=True)


def _point_conv(g1, spt, maskf, KPt, Wk, a0, c0, bk, sblk=256):
    B, S, K, _ = g1.shape
    CMID = Wk.shape[1]
    KP_N = Wk.shape[0]
    grid = (B, S // sblk)
    return pl.pallas_call(
        _k2_body,
        grid=grid,
        in_specs=[
            pl.BlockSpec((1, sblk, K, 128), lambda b, j: (b, j, 0, 0)),
            pl.BlockSpec((1, sblk, 3), lambda b, j: (b, j, 0)),
            pl.BlockSpec((1, sblk, K), lambda b, j: (b, j, 0)),
            pl.BlockSpec((3, KP_N), lambda b, j: (0, 0)),
            pl.BlockSpec((KP_N, CMID, CMID), lambda b, j: (0, 0, 0)),
            pl.BlockSpec((1, CMID), lambda b, j: (0, 0)),
            pl.BlockSpec((1, CMID), lambda b, j: (0, 0)),
            pl.BlockSpec((1, CMID), lambda b, j: (0, 0)),
        ],
        out_specs=[
            pl.BlockSpec((1, sblk, CMID), lambda b, j: (b, j, 0)),
            pl.BlockSpec((1, CMID), lambda b, j: (0, 0)),
            pl.BlockSpec((1, CMID), lambda b, j: (0, 0)),
        ],
        out_shape=[
            jax.ShapeDtypeStruct((B, S, CMID), jnp.float32),
            jax.ShapeDtypeStruct((1, CMID), jnp.float32),
            jax.ShapeDtypeStruct((1, CMID), jnp.float32),
        ],
    )(g1, spt, maskf, KPt, Wk,
      a0.reshape(1, -1), c0.reshape(1, -1), bk.reshape(1, -1))


def _k3_body(y1_ref, xg_ref, m_ref, w2_ref, a1_ref, c1_ref, b2_ref,
             y2t_ref, xs_ref, s1_ref, s2_ref):
    b = pl.program_id(0)
    j = pl.program_id(1)
    h1 = jnp.maximum(y1_ref[0] * a1_ref[...] + c1_ref[...], 0.0)
    y2 = lax.dot_general(h1, w2_ref[...], (((1,), (1,)), ((), ())),
                         preferred_element_type=jnp.float32)
    y2 = y2 + b2_ref[...]
    y2t_ref[0] = y2

    xg = xg_ref[0]
    m = m_ref[0][:, :, None]
    xm = jnp.where(m > 0.0, xg, -1e9)
    xs = jnp.max(xm, axis=1)
    xs = jnp.where(xs <= -1e9 + 1.0, 0.0, xs)
    xs_ref[0] = xs

    @pl.when((b == 0) & (j == 0))
    def _():
        s1_ref[...] = jnp.zeros_like(s1_ref)
        s2_ref[...] = jnp.zeros_like(s2_ref)

    s1_ref[...] += jnp.sum(y2, axis=0, keepdims=True)
    s2_ref[...] += jnp.sum(y2 * y2, axis=0, keepdims=True)


def _conv2_pool(y1t, xg, maskf, W2, a1, c1, b2, sblk=256):
    B, S, CMID = y1t.shape
    COUT = W2.shape[0]
    K = xg.shape[2]
    grid = (B, S // sblk)
    return pl.pallas_call(
        _k3_body,
        grid=grid,
        in_specs=[
            pl.BlockSpec((1, sblk, CMID), lambda b, j: (b, j, 0)),
            pl.BlockSpec((1, sblk, K, COUT), lambda b, j: (b, j, 0, 0)),
            pl.BlockSpec((1, sblk, K), lambda b, j: (b, j, 0)),
            pl.BlockSpec((COUT, CMID), lambda b, j: (0, 0)),
            pl.BlockSpec((1, CMID), lambda b, j: (0, 0)),
            pl.BlockSpec((1, CMID), lambda b, j: (0, 0)),
            pl.BlockSpec((1, COUT), lambda b, j: (0, 0)),
        ],
        out_specs=[
            pl.BlockSpec((1, sblk, COUT), lambda b, j: (b, j, 0)),
            pl.BlockSpec((1, sblk, COUT), lambda b, j: (b, j, 0)),
            pl.BlockSpec((1, COUT), lambda b, j: (0, 0)),
            pl.BlockSpec((1, COUT), lambda b, j: (0, 0)),
        ],
        out_shape=[
            jax.ShapeDtypeStruct((B, S, COUT), jnp.float32),
            jax.ShapeDtypeStruct((B, S, COUT), jnp.float32),
            jax.ShapeDtypeStruct((1, COUT), jnp.float32),
            jax.ShapeDtypeStruct((1, COUT), jnp.float32),
        ],
    )(y1t, xg, maskf, W2, a1.reshape(1, -1), c1.reshape(1, -1),
      b2.reshape(1, -1))


def _k4_body(y2_ref, xs_ref, a2_ref, c2_ref, out_ref):
    y = y2_ref[0] * a2_ref[...] + c2_ref[...] + xs_ref[0]
    out_ref[0] = jnp.maximum(y, 0.0).T


def _finalize(y2t, xs, a2, c2, sblk=512):
    B, S, COUT = y2t.shape
    grid = (B, S // sblk)
    return pl.pallas_call(
        _k4_body,
        grid=grid,
        in_specs=[
            pl.BlockSpec((1, sblk, COUT), lambda b, j: (b, j, 0)),
            pl.BlockSpec((1, sblk, COUT), lambda b, j: (b, j, 0)),
            pl.BlockSpec((1, COUT), lambda b, j: (0, 0)),
            pl.BlockSpec((1, COUT), lambda b, j: (0, 0)),
        ],
        out_specs=pl.BlockSpec((1, COUT, sblk), lambda b, j: (b, 0, j)),
        out_shape=jax.ShapeDtypeStruct((B, COUT, S), jnp.float32),
    )(y2t, xs, a2.reshape(1, -1), c2.reshape(1, -1))


def _bn_affine(s1, s2, cnt, g, be):
    m = s1.reshape(-1) / cnt
    v = s2.reshape(-1) / cnt - m * m
    a = g * lax.rsqrt(v + EPS)
    c = be - m * a
    return a, c


@jax.jit
def kernel(x, pos, support_points, neighbors_indices, mask_indices,
           W0, b0, g0, be0, KP, Wk, bk, g1, be1, W2, b2, g2, be2):
    B, CIN, N = x.shape
    S, K = neighbors_indices.shape[1:]
    maskf = mask_indices.astype(jnp.float32)

    post = jnp.transpose(pos, (0, 2, 1))
    y0t, xt, s1_0, s2_0 = _conv0_stats(x, post, W0, b0)
    a0, c0 = _bn_affine(s1_0, s2_0, B * N, g0, be0)

    nbr = neighbors_indices
    idx_flat = (nbr + (jnp.arange(B, dtype=nbr.dtype) * N)[:, None, None])
    idx2d = idx_flat.reshape(-1, _CHUNK)
    gx = _sc_gather(xt.reshape(B * N, CIN), idx2d)
    gf = _sc_gather(y0t.reshape(B * N, 128), idx2d)
    gf = gf.reshape(B, S, K, 128)
    xg = gx.reshape(B, S, K, CIN)
    spt = jnp.transpose(support_points, (0, 2, 1))

    y1t, s1_1, s2_1 = _point_conv(gf, spt, maskf, KP.T, Wk, a0, c0, bk)
    a1, c1 = _bn_affine(s1_1, s2_1, B * S, g1, be1)

    y2t, xs, s1_2, s2_2 = _conv2_pool(y1t, xg, maskf, W2, a1, c1, b2)
    a2, c2 = _bn_affine(s1_2, s2_2, B * S, g2, be2)

    out = _finalize(y2t, xs, a2, c2)
    return (out, support_points, neighbors_indices, mask_indices)


# speedup vs baseline: 1.1956x; 1.0594x over previous
import functools
import jax
import jax.numpy as jnp
from jax import lax
from jax.experimental import pallas as pl
from jax.experimental.pallas import tpu as pltpu, tpu_sc as plsc

SIGMA = 1.0
EPS = 1e-5

_CHUNK = 128


def _sc_gather(t, idx2d):
    n_rows = idx2d.shape[0] * idx2d.shape[1]
    D = t.shape[1]
    info = plsc.get_sparse_core_info()
    nw = info.num_cores * info.num_subcores
    cpw = idx2d.shape[0] // nw

    @functools.partial(
        pl.kernel,
        mesh=plsc.VectorSubcoreMesh(core_axis_name="c", subcore_axis_name="s"),
        out_type=jax.ShapeDtypeStruct((n_rows, D), jnp.float32),
        scratch_types=[
            pltpu.VMEM((cpw, _CHUNK), jnp.int32),
            pltpu.VMEM((_CHUNK, D), jnp.float32),
            pltpu.VMEM((_CHUNK, D), jnp.float32),
            pltpu.SemaphoreType.DMA,
            pltpu.SemaphoreType.DMA,
        ],
    )
    def gather(t_h, idx_h, o_h, idx_v, ba, bb, sema, semb):
        wid = lax.axis_index("s") * info.num_cores + lax.axis_index("c")
        pltpu.sync_copy(idx_h.at[pl.ds(wid * cpw, cpw)], idx_v)

        def fire(j, buf, sem):
            pltpu.make_async_copy(t_h.at[idx_v.at[j]], buf, sem).start()

        def drain(j, buf, sem):
            pltpu.make_async_copy(t_h.at[idx_v.at[j]], buf, sem).wait()
            row0 = (wid * cpw + j) * _CHUNK
            pltpu.sync_copy(buf, o_h.at[pl.ds(row0, _CHUNK)])

        fire(0, ba, sema)

        def pair(i, carry):
            fire(2 * i + 1, bb, semb)
            drain(2 * i, ba, sema)

            @pl.when(i < cpw // 2 - 1)
            def _():
                fire(2 * i + 2, ba, sema)

            drain(2 * i + 1, bb, semb)
            return carry

        lax.fori_loop(0, cpw // 2, pair, 0)

    return gather(t, idx2d)


def _k1_body(x_ref, pt_ref, w0_ref, b0_ref, y0t_ref, xt_ref, s1_ref, s2_ref):
    b = pl.program_id(0)
    j = pl.program_id(1)
    xb = x_ref[0]
    w0 = w0_ref[...]
    y = lax.dot_general(xb, w0, (((0,), (1,)), ((), ())),
                        preferred_element_type=jnp.float32)
    y = y + b0_ref[...]
    nblk, cmid = y.shape
    pad = jnp.zeros((nblk, 128 - cmid - 3), jnp.float32)
    y0t_ref[0] = jnp.concatenate([y, pt_ref[0], pad], axis=1)
    xt_ref[0] = xb.T

    @pl.when((b == 0) & (j == 0))
    def _():
        s1_ref[...] = jnp.zeros_like(s1_ref)
        s2_ref[...] = jnp.zeros_like(s2_ref)

    s1_ref[...] += jnp.sum(y, axis=0, keepdims=True)
    s2_ref[...] += jnp.sum(y * y, axis=0, keepdims=True)


def _conv0_stats(x, post, W0, b0, nblk=1024):
    B, CIN, N = x.shape
    CMID = W0.shape[0]
    grid = (B, N // nblk)
    return pl.pallas_call(
        _k1_body,
        grid=grid,
        in_specs=[
            pl.BlockSpec((1, CIN, nblk), lambda b, j: (b, 0, j)),
            pl.BlockSpec((1, nblk, 3), lambda b, j: (b, j, 0)),
            pl.BlockSpec((CMID, CIN), lambda b, j: (0, 0)),
            pl.BlockSpec((1, CMID), lambda b, j: (0, 0)),
        ],
        out_specs=[
            pl.BlockSpec((1, nblk, 128), lambda b, j: (b, j, 0)),
            pl.BlockSpec((1, nblk, CIN), lambda b, j: (b, j, 0)),
            pl.BlockSpec((1, CMID), lambda b, j: (0, 0)),
            pl.BlockSpec((1, CMID), lambda b, j: (0, 0)),
        ],
        out_shape=[
            jax.ShapeDtypeStruct((B, N, 128), jnp.float32),
            jax.ShapeDtypeStruct((B, N, CIN), jnp.float32),
            jax.ShapeDtypeStruct((1, CMID), jnp.float32),
            jax.ShapeDtypeStruct((1, CMID), jnp.float32),
        ],
    )(x, post, W0, b0.reshape(1, -1))


def _k2_body(f_ref, sp_ref, m_ref, kpt_ref, wk_ref,
             a0_ref, c0_ref, bk_ref, y1t_ref, s1_ref, s2_ref):
    b = pl.program_id(0)
    j = pl.program_id(1)
    KP_N = wk_ref.shape[0]
    CMID = wk_ref.shape[1]
    g = f_ref[0]
    sblk, K = g.shape[0], g.shape[1]
    f_raw = lax.slice(g, (0, 0, 0), (sblk, K, CMID))
    a0 = a0_ref[...][:, None, :]
    c0 = c0_ref[...][:, None, :]
    f = jnp.maximum(f_raw * a0 + c0, 0.0)

    kx = kpt_ref[0][None, None, :]
    ky = kpt_ref[1][None, None, :]
    kz = kpt_ref[2][None, None, :]
    sp = sp_ref[0]
    rx = (lax.slice(g, (0, 0, CMID), (sblk, K, CMID + 1))
          - lax.slice(sp, (0, 0), (sblk, 1))[:, :, None])
    ry = (lax.slice(g, (0, 0, CMID + 1), (sblk, K, CMID + 2))
          - lax.slice(sp, (0, 1), (sblk, 2))[:, :, None])
    rz = (lax.slice(g, (0, 0, CMID + 2), (sblk, K, CMID + 3))
          - lax.slice(sp, (0, 2), (sblk, 3))[:, :, None])
    d2 = (rx - kx) ** 2 + (ry - ky) ** 2 + (rz - kz) ** 2
    d = jnp.sqrt(d2 + 1e-12)
    w = jnp.maximum(1.0 - d / SIGMA, 0.0) * m_ref[0][:, :, None]

    gpc = lax.dot_general(w, f, (((1,), (1,)), ((0,), (0,))),
                          preferred_element_type=jnp.float32)
    acc = jnp.zeros((sblk, CMID), jnp.float32)
    for p in range(KP_N):
        gp = lax.slice(gpc, (0, p, 0), (sblk, p + 1, CMID)).reshape(sblk, CMID)
        acc = acc + jnp.dot(gp, wk_ref[p],
                            preferred_element_type=jnp.float32)
    y1 = acc + bk_ref[...]
    y1t_ref[0] = y1

    @pl.when((b == 0) & (j == 0))
    def _():
        s1_ref[...] = jnp.zeros_like(s1_ref)
        s2_ref[...] = jnp.zeros_like(s2_ref)

    s1_ref[...] += jnp.sum(y1, axis=0, keepdims=True)
    s2_ref[...] += jnp.sum(y1 * y1, axis=0, keepdims=True)


def _point_conv(g1, spt, maskf, KPt, Wk, a0, c0, bk, sblk=512):
    B, S, K, _ = g1.shape
    CMID = Wk.shape[1]
    KP_N = Wk.shape[0]
    grid = (B, S // sblk)
    return pl.pallas_call(
        _k2_body,
        grid=grid,
        in_specs=[
            pl.BlockSpec((1, sblk, K, 128), lambda b, j: (b, j, 0, 0)),
            pl.BlockSpec((1, sblk, 3), lambda b, j: (b, j, 0)),
            pl.BlockSpec((1, sblk, K), lambda b, j: (b, j, 0)),
            pl.BlockSpec((3, KP_N), lambda b, j: (0, 0)),
            pl.BlockSpec((KP_N, CMID, CMID), lambda b, j: (0, 0, 0)),
            pl.BlockSpec((1, CMID), lambda b, j: (0, 0)),
            pl.BlockSpec((1, CMID), lambda b, j: (0, 0)),
            pl.BlockSpec((1, CMID), lambda b, j: (0, 0)),
        ],
        out_specs=[
            pl.BlockSpec((1, sblk, CMID), lambda b, j: (b, j, 0)),
            pl.BlockSpec((1, CMID), lambda b, j: (0, 0)),
            pl.BlockSpec((1, CMID), lambda b, j: (0, 0)),
        ],
        out_shape=[
            jax.ShapeDtypeStruct((B, S, CMID), jnp.float32),
            jax.ShapeDtypeStruct((1, CMID), jnp.float32),
            jax.ShapeDtypeStruct((1, CMID), jnp.float32),
        ],
    )(g1, spt, maskf, KPt, Wk,
      a0.reshape(1, -1), c0.reshape(1, -1), bk.reshape(1, -1))


def _k3_body(y1_ref, xg_ref, m_ref, w2_ref, a1_ref, c1_ref, b2_ref,
             y2t_ref, xs_ref, s1_ref, s2_ref):
    b = pl.program_id(0)
    j = pl.program_id(1)
    h1 = jnp.maximum(y1_ref[0] * a1_ref[...] + c1_ref[...], 0.0)
    y2 = lax.dot_general(h1, w2_ref[...], (((1,), (1,)), ((), ())),
                         preferred_element_type=jnp.float32)
    y2 = y2 + b2_ref[...]
    y2t_ref[0] = y2

    xg = xg_ref[0]
    m = m_ref[0][:, :, None]
    xm = jnp.where(m > 0.0, xg, -1e9)
    xs = jnp.max(xm, axis=1)
    xs = jnp.where(xs <= -1e9 + 1.0, 0.0, xs)
    xs_ref[0] = xs

    @pl.when((b == 0) & (j == 0))
    def _():
        s1_ref[...] = jnp.zeros_like(s1_ref)
        s2_ref[...] = jnp.zeros_like(s2_ref)

    s1_ref[...] += jnp.sum(y2, axis=0, keepdims=True)
    s2_ref[...] += jnp.sum(y2 * y2, axis=0, keepdims=True)


def _conv2_pool(y1t, xg, maskf, W2, a1, c1, b2, sblk=256):
    B, S, CMID = y1t.shape
    COUT = W2.shape[0]
    K = xg.shape[2]
    grid = (B, S // sblk)
    return pl.pallas_call(
        _k3_body,
        grid=grid,
        in_specs=[
            pl.BlockSpec((1, sblk, CMID), lambda b, j: (b, j, 0)),
            pl.BlockSpec((1, sblk, K, COUT), lambda b, j: (b, j, 0, 0)),
            pl.BlockSpec((1, sblk, K), lambda b, j: (b, j, 0)),
            pl.BlockSpec((COUT, CMID), lambda b, j: (0, 0)),
            pl.BlockSpec((1, CMID), lambda b, j: (0, 0)),
            pl.BlockSpec((1, CMID), lambda b, j: (0, 0)),
            pl.BlockSpec((1, COUT), lambda b, j: (0, 0)),
        ],
        out_specs=[
            pl.BlockSpec((1, sblk, COUT), lambda b, j: (b, j, 0)),
            pl.BlockSpec((1, sblk, COUT), lambda b, j: (b, j, 0)),
            pl.BlockSpec((1, COUT), lambda b, j: (0, 0)),
            pl.BlockSpec((1, COUT), lambda b, j: (0, 0)),
        ],
        out_shape=[
            jax.ShapeDtypeStruct((B, S, COUT), jnp.float32),
            jax.ShapeDtypeStruct((B, S, COUT), jnp.float32),
            jax.ShapeDtypeStruct((1, COUT), jnp.float32),
            jax.ShapeDtypeStruct((1, COUT), jnp.float32),
        ],
    )(y1t, xg, maskf, W2, a1.reshape(1, -1), c1.reshape(1, -1),
      b2.reshape(1, -1))


def _k4_body(y2_ref, xs_ref, a2_ref, c2_ref, out_ref):
    y = y2_ref[0] * a2_ref[...] + c2_ref[...] + xs_ref[0]
    out_ref[0] = jnp.maximum(y, 0.0).T


def _finalize(y2t, xs, a2, c2, sblk=512):
    B, S, COUT = y2t.shape
    grid = (B, S // sblk)
    return pl.pallas_call(
        _k4_body,
        grid=grid,
        in_specs=[
            pl.BlockSpec((1, sblk, COUT), lambda b, j: (b, j, 0)),
            pl.BlockSpec((1, sblk, COUT), lambda b, j: (b, j, 0)),
            pl.BlockSpec((1, COUT), lambda b, j: (0, 0)),
            pl.BlockSpec((1, COUT), lambda b, j: (0, 0)),
        ],
        out_specs=pl.BlockSpec((1, COUT, sblk), lambda b, j: (b, 0, j)),
        out_shape=jax.ShapeDtypeStruct((B, COUT, S), jnp.float32),
    )(y2t, xs, a2.reshape(1, -1), c2.reshape(1, -1))


def _bn_affine(s1, s2, cnt, g, be):
    m = s1.reshape(-1) / cnt
    v = s2.reshape(-1) / cnt - m * m
    a = g * lax.rsqrt(v + EPS)
    c = be - m * a
    return a, c


@jax.jit
def kernel(x, pos, support_points, neighbors_indices, mask_indices,
           W0, b0, g0, be0, KP, Wk, bk, g1, be1, W2, b2, g2, be2):
    B, CIN, N = x.shape
    S, K = neighbors_indices.shape[1:]
    maskf = mask_indices.astype(jnp.float32)

    post = jnp.transpose(pos, (0, 2, 1))
    y0t, xt, s1_0, s2_0 = _conv0_stats(x, post, W0, b0)
    a0, c0 = _bn_affine(s1_0, s2_0, B * N, g0, be0)

    nbr = neighbors_indices
    idx_flat = (nbr + (jnp.arange(B, dtype=nbr.dtype) * N)[:, None, None])
    idx2d = idx_flat.reshape(-1, _CHUNK)
    gx = _sc_gather(xt.reshape(B * N, CIN), idx2d)
    gf = _sc_gather(y0t.reshape(B * N, 128), idx2d)
    gf = gf.reshape(B, S, K, 128)
    xg = gx.reshape(B, S, K, CIN)
    spt = jnp.transpose(support_points, (0, 2, 1))

    y1t, s1_1, s2_1 = _point_conv(gf, spt, maskf, KP.T, Wk, a0, c0, bk)
    a1, c1 = _bn_affine(s1_1, s2_1, B * S, g1, be1)

    y2t, xs, s1_2, s2_2 = _conv2_pool(y1t, xg, maskf, W2, a1, c1, b2)
    a2, c2 = _bn_affine(s1_2, s2_2, B * S, g2, be2)

    out = _finalize(y2t, xs, a2, c2)
    return (out, support_points, neighbors_indices, mask_indices)
